# baseline (speedup 1.0000x reference)
import dataclasses
import functools

import jax
import jax.numpy as jnp
from jax import lax
from jax.experimental import pallas as pl
from jax.experimental.pallas import tpu as pltpu
from jax.experimental.pallas import tpu_sc as plsc

N = 10000
E = 320000
F = 128
OUT = 32
G = 128

N_PAD = 10240
NCORES = 2
NSUB = 16
NQ = 4
NFG = 8
EQ = E // NQ
CHUNK_B = 8000
NCH = EQ // CHUNK_B
CHUNK_A = 4000
EA = E // NSUB
SLICE = N_PAD // NSUB
BLK = 512
NB = N_PAD // BLK


def _xwt_body(x_ref, w_ref, o_ref):
    o_ref[...] = lax.dot_general(
        w_ref[...], x_ref[...],
        dimension_numbers=(((0,), (1,)), ((), ())),
        preferred_element_type=jnp.float32)


def _tc1(x, W):
    return pl.pallas_call(
        _xwt_body,
        grid=(NB,),
        in_specs=[
            pl.BlockSpec((BLK, F), lambda i: (i, 0)),
            pl.BlockSpec((F, OUT), lambda i: (0, 0)),
        ],
        out_specs=pl.BlockSpec((OUT, BLK), lambda i: (0, i)),
        out_shape=jax.ShapeDtypeStruct((OUT, N_PAD), jnp.float32),
    )(x, W)


def _rsqrt16(v):
    ii = plsc.bitcast(v, jnp.int32)
    ii = jnp.int32(0x5F3759DF) - lax.shift_right_arithmetic(ii, 1)
    g = plsc.bitcast(ii, jnp.float32)
    g = g * (1.5 - 0.5 * v * g * g)
    g = g * (1.5 - 0.5 * v * g * g)
    g = g * (1.5 - 0.5 * v * g * g)
    return g


def _sca_body(dst_hbm, dis_hbm, cnt, hbufs, tmps, accv, cnt_sh, sems):
    c = lax.axis_index("c")
    s = lax.axis_index("s")

    zeros16 = jnp.zeros((16,), jnp.float32)
    ones16 = jnp.ones((16,), jnp.float32)

    NCA = EA // CHUNK_A
    pltpu.async_copy(dst_hbm.at[pl.ds(s * EA, CHUNK_A)], hbufs[0], sems[0])

    @pl.loop(0, N_PAD, step=16, unroll=4)
    def _(i):
        cnt[pl.ds(i, 16)] = zeros16

    for k in range(NCA):
        b = k % 2
        pltpu.make_async_copy(
            dst_hbm.at[pl.ds(0, CHUNK_A)], hbufs[b], sems[b]).wait()
        if k + 1 < NCA:
            pltpu.async_copy(
                dst_hbm.at[pl.ds(s * EA + (k + 1) * CHUNK_A, CHUNK_A)],
                hbufs[1 - b], sems[1 - b])

        @pl.loop(0, CHUNK_A, step=16, unroll=4)
        def _(g):
            idx = hbufs[b][pl.ds(g, 16)]
            plsc.addupdate_scatter(cnt, [idx], ones16)

    pltpu.sync_copy(cnt, cnt_sh.at[pl.ds(s * N_PAD, N_PAD)])
    plsc.subcore_barrier()

    pltpu.async_copy(cnt_sh.at[pl.ds(s * SLICE, SLICE)], tmps[0], sems[0])

    @pl.loop(0, SLICE, step=16, unroll=4)
    def _(i):
        accv[pl.ds(i, 16)] = zeros16

    for t in range(NSUB):
        b = t % 2
        pltpu.make_async_copy(
            cnt_sh.at[pl.ds(0, SLICE)], tmps[b], sems[b]).wait()
        if t + 1 < NSUB:
            pltpu.async_copy(
                cnt_sh.at[pl.ds((t + 1) * N_PAD + s * SLICE, SLICE)],
                tmps[1 - b], sems[1 - b])

        @pl.loop(0, SLICE, step=16, unroll=4)
        def _(i):
            accv[pl.ds(i, 16)] += tmps[b][pl.ds(i, 16)]

    @pl.loop(0, SLICE, step=16, unroll=2)
    def _(i):
        accv[pl.ds(i, 16)] = _rsqrt16(accv[pl.ds(i, 16)] + 1.0)

    @pl.when(c == 0)
    def _():
        pltpu.sync_copy(accv, dis_hbm.at[pl.ds(s * SLICE, SLICE)])


def _sca_call(dst):
    mesh = plsc.VectorSubcoreMesh(core_axis_name="c", subcore_axis_name="s")
    cp = pltpu.CompilerParams()
    if "needs_layout_passes" in pltpu.CompilerParams.__dataclass_fields__:
        cp = dataclasses.replace(cp, needs_layout_passes=False)
    kern = pl.kernel(
        _sca_body,
        out_type=jax.ShapeDtypeStruct((N_PAD,), jnp.float32),
        mesh=mesh,
        scratch_types=[
            pltpu.VMEM((N_PAD,), jnp.float32),
            [pltpu.VMEM((CHUNK_A,), jnp.int32) for _ in range(2)],
            [pltpu.VMEM((SLICE,), jnp.float32) for _ in range(2)],
            pltpu.VMEM((SLICE,), jnp.float32),
            pltpu.VMEM_SHARED((NSUB * N_PAD,), jnp.float32),
            [pltpu.SemaphoreType.DMA for _ in range(2)],
        ],
        compiler_params=cp,
    )
    return kern(dst)


def _scb_body(xwt_hbm, src_hbm, dst_hbm, dis_hbm, spart_hbm,
              y_refs, s_refs, dis_v, srcbs, dstbs, sems):
    c = lax.axis_index("c")
    s = lax.axis_index("s")
    wid = c * NSUB + s
    fg = wid // NQ
    q = wid % NQ

    zeros16 = jnp.zeros((16,), jnp.float32)

    for j in range(4):
        pltpu.async_copy(xwt_hbm.at[pl.ds((fg * 4 + j) * N_PAD, N_PAD)],
                         y_refs[j], sems[2])
    pltpu.async_copy(dis_hbm, dis_v, sems[2])
    pltpu.async_copy(src_hbm.at[pl.ds(q * EQ, CHUNK_B)], srcbs[0], sems[0])
    pltpu.async_copy(dst_hbm.at[pl.ds(q * EQ, CHUNK_B)], dstbs[0], sems[0])

    with jax.named_scope("scB_stage"):
        @pl.loop(0, N_PAD, step=16, unroll=4)
        def _(i):
            for j in range(4):
                s_refs[j][pl.ds(i, 16)] = zeros16

        for j in range(4):
            pltpu.make_async_copy(
                xwt_hbm.at[pl.ds(0, N_PAD)], y_refs[j], sems[2]).wait()
        pltpu.make_async_copy(dis_hbm, dis_v, sems[2]).wait()

        @pl.loop(0, N_PAD, step=16, unroll=4)
        def _(i):
            d = dis_v[pl.ds(i, 16)]
            for j in range(4):
                y_refs[j][pl.ds(i, 16)] = y_refs[j][pl.ds(i, 16)] * d

    with jax.named_scope("scB_edges"):
        @pl.loop(0, NCH, step=2)
        def _(ci):
            for b in range(2):
                cc = ci + b
                pltpu.make_async_copy(
                    src_hbm.at[pl.ds(0, CHUNK_B)], srcbs[b], sems[b]).wait()
                pltpu.make_async_copy(
                    dst_hbm.at[pl.ds(0, CHUNK_B)], dstbs[b], sems[b]).wait()

                @pl.when(cc + 1 < NCH)
                def _():
                    nbase = q * EQ + (cc + 1) * CHUNK_B
                    pltpu.async_copy(src_hbm.at[pl.ds(nbase, CHUNK_B)],
                                     srcbs[1 - b], sems[1 - b])
                    pltpu.async_copy(dst_hbm.at[pl.ds(nbase, CHUNK_B)],
                                     dstbs[1 - b], sems[1 - b])

                @pl.loop(0, CHUNK_B, step=16, unroll=4)
                def _(g):
                    si = srcbs[b][pl.ds(g, 16)]
                    di = dstbs[b][pl.ds(g, 16)]
                    for j in range(4):
                        v = plsc.load_gather(y_refs[j], [si])
                        plsc.addupdate_scatter(s_refs[j], [di], v)

    with jax.named_scope("scB_store"):
        for j in range(4):
            pltpu.async_copy(s_refs[j],
                             spart_hbm.at[pl.ds((q * OUT + fg * 4 + j) * N_PAD,
                                                N_PAD)],
                             sems[2])
        for j in range(4):
            pltpu.make_async_copy(
                s_refs[j], spart_hbm.at[pl.ds(0, N_PAD)], sems[2]).wait()


def _scb_call(xwt_flat, src, dst, dis):
    mesh = plsc.VectorSubcoreMesh(core_axis_name="c", subcore_axis_name="s")
    cp = pltpu.CompilerParams()
    if "needs_layout_passes" in pltpu.CompilerParams.__dataclass_fields__:
        cp = dataclasses.replace(cp, needs_layout_passes=False)
    kern = pl.kernel(
        _scb_body,
        out_type=jax.ShapeDtypeStruct((NQ * OUT * N_PAD,), jnp.float32),
        mesh=mesh,
        scratch_types=[
            [pltpu.VMEM((N_PAD,), jnp.float32) for _ in range(4)],
            [pltpu.VMEM((N_PAD,), jnp.float32) for _ in range(4)],
            pltpu.VMEM((N_PAD,), jnp.float32),
            [pltpu.VMEM((CHUNK_B,), jnp.int32) for _ in range(2)],
            [pltpu.VMEM((CHUNK_B,), jnp.int32) for _ in range(2)],
            [pltpu.SemaphoreType.DMA for _ in range(3)],
        ],
        compiler_params=cp,
    )
    return kern(xwt_flat, src, dst, dis)


def _tc2_body(spart_ref, xwt_ref, dis_ref, batch_ref, b_ref, o_ref,
              pooled_acc, cnt_acc):
    i = pl.program_id(0)

    @pl.when(i == 0)
    def _():
        pooled_acc[...] = jnp.zeros_like(pooled_acc)
        cnt_acc[...] = jnp.zeros_like(cnt_acc)

    s_blk = (spart_ref[0] + spart_ref[1] + spart_ref[2] + spart_ref[3])
    dis_b = dis_ref[0]
    agg = dis_b * (s_blk + xwt_ref[...] * dis_b) + b_ref[:, :1]
    h = jnp.tanh(agg)

    col = i * BLK + lax.broadcasted_iota(jnp.int32, (1, BLK), 1)
    valid = col < N
    h = jnp.where(valid, h, 0.0)

    bat = batch_ref[0]
    rows = lax.broadcasted_iota(jnp.int32, (G, BLK), 0)
    onehot_t = jnp.where((rows == bat) & valid, 1.0, 0.0)

    pooled_acc[...] += lax.dot_general(
        h, onehot_t, dimension_numbers=(((1,), (1,)), ((), ())),
        preferred_element_type=jnp.float32)
    cnt_acc[...] += lax.dot_general(
        jnp.ones((1, BLK), jnp.float32), onehot_t,
        dimension_numbers=(((1,), (1,)), ((), ())),
        preferred_element_type=jnp.float32)

    @pl.when(i == NB - 1)
    def _():
        pm = pooled_acc[...] / jnp.maximum(cnt_acc[...], 1.0)
        o_ref[...] = jnp.tanh(pm).T


def _tc2(spart, xwt, dis3, batch3, b_bcast):
    return pl.pallas_call(
        _tc2_body,
        grid=(NB,),
        in_specs=[
            pl.BlockSpec((NQ, OUT, BLK), lambda i: (0, 0, i)),
            pl.BlockSpec((OUT, BLK), lambda i: (0, i)),
            pl.BlockSpec((1, 1, BLK), lambda i: (i, 0, 0)),
            pl.BlockSpec((1, 1, BLK), lambda i: (i, 0, 0)),
            pl.BlockSpec((OUT, 128), lambda i: (0, 0)),
        ],
        out_specs=pl.BlockSpec((G, OUT), lambda i: (0, 0)),
        out_shape=jax.ShapeDtypeStruct((G, OUT), jnp.float32),
        scratch_shapes=[
            pltpu.VMEM((OUT, G), jnp.float32),
            pltpu.VMEM((1, G), jnp.float32),
        ],
    )(spart, xwt, dis3, batch3, b_bcast)


@jax.jit
def kernel(x, edge_index, batch_index, W, b):
    src = edge_index[0]
    dst = edge_index[1]

    dis = _sca_call(dst)
    xwt = _tc1(x, W)
    spart_flat = _scb_call(xwt.reshape(-1), src, dst, dis)

    spart = spart_flat.reshape(NQ, OUT, N_PAD)
    dis3 = dis.reshape(NB, 1, BLK)
    batch3 = jnp.pad(batch_index, (0, N_PAD - N)).reshape(NB, 1, BLK)
    b_bcast = jnp.broadcast_to(b[:, None], (OUT, 128))

    return _tc2(spart, xwt, dis3, batch3, b_bcast)


# speedup vs baseline: 1.0171x; 1.0171x over previous
import dataclasses
import functools

import jax
import jax.numpy as jnp
from jax import lax
from jax.experimental import pallas as pl
from jax.experimental.pallas import tpu as pltpu
from jax.experimental.pallas import tpu_sc as plsc

N = 10000
E = 320000
F = 128
OUT = 32
G = 128

N_PAD = 10240
NCORES = 2
NSUB = 16
NQ = 4
NFG = 8
EQ = E // NQ
CHUNK_B = 1600
NCH = EQ // CHUNK_B
CHUNK_A = 4000
EA = E // NSUB
SLICE = N_PAD // NSUB
BLK = 512
NB = N_PAD // BLK


def _xwt_body(x_ref, w_ref, o_ref):
    o_ref[...] = lax.dot_general(
        w_ref[...], x_ref[...],
        dimension_numbers=(((0,), (1,)), ((), ())),
        preferred_element_type=jnp.float32)


def _tc1(x, W):
    return pl.pallas_call(
        _xwt_body,
        grid=(NB,),
        in_specs=[
            pl.BlockSpec((BLK, F), lambda i: (i, 0)),
            pl.BlockSpec((F, OUT), lambda i: (0, 0)),
        ],
        out_specs=pl.BlockSpec((OUT, BLK), lambda i: (0, i)),
        out_shape=jax.ShapeDtypeStruct((OUT, N_PAD), jnp.float32),
    )(x, W)


def _rsqrt16(v):
    ii = plsc.bitcast(v, jnp.int32)
    ii = jnp.int32(0x5F3759DF) - lax.shift_right_arithmetic(ii, 1)
    g = plsc.bitcast(ii, jnp.float32)
    g = g * (1.5 - 0.5 * v * g * g)
    g = g * (1.5 - 0.5 * v * g * g)
    g = g * (1.5 - 0.5 * v * g * g)
    return g


def _sca_body(dst_hbm, dis_hbm, cnt, hbufs, tmps, accv, cnt_sh, sems):
    c = lax.axis_index("c")
    s = lax.axis_index("s")

    zeros16 = jnp.zeros((16,), jnp.float32)
    ones16 = jnp.ones((16,), jnp.float32)

    NCA = EA // CHUNK_A
    pltpu.async_copy(dst_hbm.at[pl.ds(s * EA, CHUNK_A)], hbufs[0], sems[0])

    @pl.loop(0, N_PAD, step=16, unroll=4)
    def _(i):
        cnt[pl.ds(i, 16)] = zeros16

    for k in range(NCA):
        b = k % 2
        pltpu.make_async_copy(
            dst_hbm.at[pl.ds(0, CHUNK_A)], hbufs[b], sems[b]).wait()
        if k + 1 < NCA:
            pltpu.async_copy(
                dst_hbm.at[pl.ds(s * EA + (k + 1) * CHUNK_A, CHUNK_A)],
                hbufs[1 - b], sems[1 - b])

        @pl.loop(0, CHUNK_A, step=16, unroll=4)
        def _(g):
            idx = hbufs[b][pl.ds(g, 16)]
            plsc.addupdate_scatter(cnt, [idx], ones16)

    pltpu.sync_copy(cnt, cnt_sh.at[pl.ds(s * N_PAD, N_PAD)])
    plsc.subcore_barrier()

    pltpu.async_copy(cnt_sh.at[pl.ds(s * SLICE, SLICE)], tmps[0], sems[0])

    @pl.loop(0, SLICE, step=16, unroll=4)
    def _(i):
        accv[pl.ds(i, 16)] = zeros16

    for t in range(NSUB):
        b = t % 2
        pltpu.make_async_copy(
            cnt_sh.at[pl.ds(0, SLICE)], tmps[b], sems[b]).wait()
        if t + 1 < NSUB:
            pltpu.async_copy(
                cnt_sh.at[pl.ds((t + 1) * N_PAD + s * SLICE, SLICE)],
                tmps[1 - b], sems[1 - b])

        @pl.loop(0, SLICE, step=16, unroll=4)
        def _(i):
            accv[pl.ds(i, 16)] += tmps[b][pl.ds(i, 16)]

    @pl.loop(0, SLICE, step=16, unroll=2)
    def _(i):
        accv[pl.ds(i, 16)] = _rsqrt16(accv[pl.ds(i, 16)] + 1.0)

    @pl.when(c == 0)
    def _():
        pltpu.sync_copy(accv, dis_hbm.at[pl.ds(s * SLICE, SLICE)])


def _sca_call(dst):
    mesh = plsc.VectorSubcoreMesh(core_axis_name="c", subcore_axis_name="s")
    cp = pltpu.CompilerParams()
    if "needs_layout_passes" in pltpu.CompilerParams.__dataclass_fields__:
        cp = dataclasses.replace(cp, needs_layout_passes=False)
    kern = pl.kernel(
        _sca_body,
        out_type=jax.ShapeDtypeStruct((N_PAD,), jnp.float32),
        mesh=mesh,
        scratch_types=[
            pltpu.VMEM((N_PAD,), jnp.float32),
            [pltpu.VMEM((CHUNK_A,), jnp.int32) for _ in range(2)],
            [pltpu.VMEM((SLICE,), jnp.float32) for _ in range(2)],
            pltpu.VMEM((SLICE,), jnp.float32),
            pltpu.VMEM_SHARED((NSUB * N_PAD,), jnp.float32),
            [pltpu.SemaphoreType.DMA for _ in range(2)],
        ],
        compiler_params=cp,
    )
    return kern(dst)


def _scb_body(xwt_hbm, src_hbm, dst_hbm, dis_hbm, spart_hbm,
              y_refs, s_refs, dbufs, srcbs, dstbs, sems):
    c = lax.axis_index("c")
    s = lax.axis_index("s")
    wid = c * NSUB + s
    fg = wid // NQ
    q = wid % NQ

    zeros16 = jnp.zeros((16,), jnp.float32)

    for j in range(4):
        pltpu.async_copy(xwt_hbm.at[pl.ds((fg * 4 + j) * N_PAD, N_PAD)],
                         y_refs[j], sems[2])
    pltpu.async_copy(dis_hbm.at[pl.ds(0, SLICE)], dbufs[0], sems[2])
    pltpu.async_copy(src_hbm.at[pl.ds(q * EQ, CHUNK_B)], srcbs[0], sems[0])
    pltpu.async_copy(dst_hbm.at[pl.ds(q * EQ, CHUNK_B)], dstbs[0], sems[0])

    with jax.named_scope("scB_stage"):
        @pl.loop(0, N_PAD, step=16, unroll=4)
        def _(i):
            for j in range(8):
                s_refs[j][pl.ds(i, 16)] = zeros16

        for j in range(4):
            pltpu.make_async_copy(
                xwt_hbm.at[pl.ds(0, N_PAD)], y_refs[j], sems[2]).wait()

        for t in range(NSUB):
            db = t % 2
            pltpu.make_async_copy(
                dis_hbm.at[pl.ds(0, SLICE)], dbufs[db], sems[2]).wait()
            if t + 1 < NSUB:
                pltpu.async_copy(
                    dis_hbm.at[pl.ds((t + 1) * SLICE, SLICE)],
                    dbufs[1 - db], sems[2])

            @pl.loop(0, SLICE, step=16, unroll=4)
            def _(i):
                d = dbufs[db][pl.ds(i, 16)]
                for j in range(4):
                    y_refs[j][pl.ds(t * SLICE + i, 16)] = (
                        y_refs[j][pl.ds(t * SLICE + i, 16)] * d)

    with jax.named_scope("scB_edges"):
        @pl.loop(0, NCH, step=2)
        def _(ci):
            for b in range(2):
                cc = ci + b
                pltpu.make_async_copy(
                    src_hbm.at[pl.ds(0, CHUNK_B)], srcbs[b], sems[b]).wait()
                pltpu.make_async_copy(
                    dst_hbm.at[pl.ds(0, CHUNK_B)], dstbs[b], sems[b]).wait()

                @pl.when(cc + 1 < NCH)
                def _():
                    nbase = q * EQ + (cc + 1) * CHUNK_B
                    pltpu.async_copy(src_hbm.at[pl.ds(nbase, CHUNK_B)],
                                     srcbs[1 - b], sems[1 - b])
                    pltpu.async_copy(dst_hbm.at[pl.ds(nbase, CHUNK_B)],
                                     dstbs[1 - b], sems[1 - b])

                @pl.loop(0, CHUNK_B, step=32, unroll=2)
                def _(g):
                    si0 = srcbs[b][pl.ds(g, 16)]
                    di0 = dstbs[b][pl.ds(g, 16)]
                    si1 = srcbs[b][pl.ds(g + 16, 16)]
                    di1 = dstbs[b][pl.ds(g + 16, 16)]
                    for j in range(4):
                        v0 = plsc.load_gather(y_refs[j], [si0])
                        plsc.addupdate_scatter(s_refs[j], [di0], v0)
                        v1 = plsc.load_gather(y_refs[j], [si1])
                        plsc.addupdate_scatter(s_refs[4 + j], [di1], v1)

    with jax.named_scope("scB_store"):
        for p in range(2):
            for j in range(4):
                row = ((q * 2 + p) * OUT + fg * 4 + j) * N_PAD
                pltpu.async_copy(s_refs[4 * p + j],
                                 spart_hbm.at[pl.ds(row, N_PAD)], sems[2])
        for j in range(8):
            pltpu.make_async_copy(
                s_refs[j], spart_hbm.at[pl.ds(0, N_PAD)], sems[2]).wait()


def _scb_call(xwt_flat, src, dst, dis):
    mesh = plsc.VectorSubcoreMesh(core_axis_name="c", subcore_axis_name="s")
    cp = pltpu.CompilerParams()
    if "needs_layout_passes" in pltpu.CompilerParams.__dataclass_fields__:
        cp = dataclasses.replace(cp, needs_layout_passes=False)
    kern = pl.kernel(
        _scb_body,
        out_type=jax.ShapeDtypeStruct((2 * NQ * OUT * N_PAD,), jnp.float32),
        mesh=mesh,
        scratch_types=[
            [pltpu.VMEM((N_PAD,), jnp.float32) for _ in range(4)],
            [pltpu.VMEM((N_PAD,), jnp.float32) for _ in range(8)],
            [pltpu.VMEM((SLICE,), jnp.float32) for _ in range(2)],
            [pltpu.VMEM((CHUNK_B,), jnp.int32) for _ in range(2)],
            [pltpu.VMEM((CHUNK_B,), jnp.int32) for _ in range(2)],
            [pltpu.SemaphoreType.DMA for _ in range(3)],
        ],
        compiler_params=cp,
    )
    return kern(xwt_flat, src, dst, dis)


def _tc2_body(spart_ref, xwt_ref, dis_ref, batch_ref, b_ref, o_ref,
              pooled_acc, cnt_acc):
    i = pl.program_id(0)

    @pl.when(i == 0)
    def _():
        pooled_acc[...] = jnp.zeros_like(pooled_acc)
        cnt_acc[...] = jnp.zeros_like(cnt_acc)

    s_blk = (spart_ref[0] + spart_ref[1] + spart_ref[2] + spart_ref[3]
             + spart_ref[4] + spart_ref[5] + spart_ref[6] + spart_ref[7])
    dis_b = dis_ref[0]
    agg = dis_b * (s_blk + xwt_ref[...] * dis_b) + b_ref[:, :1]
    h = jnp.tanh(agg)

    col = i * BLK + lax.broadcasted_iota(jnp.int32, (1, BLK), 1)
    valid = col < N
    h = jnp.where(valid, h, 0.0)

    bat = batch_ref[0]
    rows = lax.broadcasted_iota(jnp.int32, (G, BLK), 0)
    onehot_t = jnp.where((rows == bat) & valid, 1.0, 0.0)

    pooled_acc[...] += lax.dot_general(
        h, onehot_t, dimension_numbers=(((1,), (1,)), ((), ())),
        preferred_element_type=jnp.float32)
    cnt_acc[...] += lax.dot_general(
        jnp.ones((1, BLK), jnp.float32), onehot_t,
        dimension_numbers=(((1,), (1,)), ((), ())),
        preferred_element_type=jnp.float32)

    @pl.when(i == NB - 1)
    def _():
        pm = pooled_acc[...] / jnp.maximum(cnt_acc[...], 1.0)
        o_ref[...] = jnp.tanh(pm).T


def _tc2(spart, xwt, dis3, batch3, b_bcast):
    return pl.pallas_call(
        _tc2_body,
        grid=(NB,),
        in_specs=[
            pl.BlockSpec((2 * NQ, OUT, BLK), lambda i: (0, 0, i)),
            pl.BlockSpec((OUT, BLK), lambda i: (0, i)),
            pl.BlockSpec((1, 1, BLK), lambda i: (i, 0, 0)),
            pl.BlockSpec((1, 1, BLK), lambda i: (i, 0, 0)),
            pl.BlockSpec((OUT, 128), lambda i: (0, 0)),
        ],
        out_specs=pl.BlockSpec((G, OUT), lambda i: (0, 0)),
        out_shape=jax.ShapeDtypeStruct((G, OUT), jnp.float32),
        scratch_shapes=[
            pltpu.VMEM((OUT, G), jnp.float32),
            pltpu.VMEM((1, G), jnp.float32),
        ],
    )(spart, xwt, dis3, batch3, b_bcast)


@jax.jit
def kernel(x, edge_index, batch_index, W, b):
    src = edge_index[0]
    dst = edge_index[1]

    dis = _sca_call(dst)
    xwt = _tc1(x, W)
    spart_flat = _scb_call(xwt.reshape(-1), src, dst, dis)

    spart = spart_flat.reshape(2 * NQ, OUT, N_PAD)
    dis3 = dis.reshape(NB, 1, BLK)
    batch3 = jnp.pad(batch_index, (0, N_PAD - N)).reshape(NB, 1, BLK)
    b_bcast = jnp.broadcast_to(b[:, None], (OUT, 128))

    return _tc2(spart, xwt, dis3, batch3, b_bcast)
